# baseline (device time: 494928 ns/iter reference)
import jax
import jax.numpy as jnp
from jax import lax
from jax.experimental import pallas as pl
from jax.experimental.pallas import tpu as pltpu

N_DEV = 8
SCALE = 0.08838834764831843
EPS = 1e-5
BLK = 512


def _ln(h):
    m = jnp.mean(h, axis=-1, keepdims=True)
    v = jnp.mean((h - m) ** 2, axis=-1, keepdims=True)
    return (h - m) * lax.rsqrt(v + EPS)


def _qkv_call(x0, scale_row, shift_row, Wq, Wk, Wv):
    S, D = x0.shape
    Dl = Wq.shape[1]

    def body(x_ref, sc_ref, sh_ref, wq_ref, wk_ref, wv_ref,
             q_ref, k_ref, v_ref):
        x = _ln(x_ref[...]) * sc_ref[...] + sh_ref[...]
        x = x.astype(jnp.bfloat16)
        q_ref[...] = (
            jnp.dot(x, wq_ref[...], preferred_element_type=jnp.float32)
            * SCALE
        ).astype(jnp.bfloat16)
        k_ref[...] = jnp.dot(
            x, wk_ref[...], preferred_element_type=jnp.float32
        ).astype(jnp.bfloat16)
        v_ref[...] = jnp.dot(
            x, wv_ref[...], preferred_element_type=jnp.float32
        ).astype(jnp.bfloat16)

    w_spec = pl.BlockSpec((D, Dl), lambda i: (0, 0))
    row_spec = pl.BlockSpec((BLK, D), lambda i: (i, 0))
    vec_spec = pl.BlockSpec((1, D), lambda i: (0, 0))
    out_spec = pl.BlockSpec((BLK, Dl), lambda i: (i, 0))
    out_shape = jax.ShapeDtypeStruct((S, Dl), jnp.bfloat16)
    return pl.pallas_call(
        body,
        grid=(S // BLK,),
        in_specs=[row_spec, vec_spec, vec_spec, w_spec, w_spec, w_spec],
        out_specs=(out_spec, out_spec, out_spec),
        out_shape=(out_shape, out_shape, out_shape),
    )(x0, scale_row, shift_row, Wq, Wk, Wv)


def _attn_call(Q, K, V):
    S, Dl = Q.shape
    Dh = 128
    H = Dl // Dh

    def body(q_ref, k_ref, v_ref, o_ref):
        q = q_ref[...]
        k = k_ref[...]
        v = v_ref[...]
        s = lax.dot_general(
            q, k, (((1,), (1,)), ((), ())),
            preferred_element_type=jnp.float32,
        )
        p = jnp.exp(s).astype(jnp.bfloat16)
        l = jnp.sum(p, axis=-1, keepdims=True, dtype=jnp.float32)
        o = jnp.dot(p, v, preferred_element_type=jnp.float32)
        o_ref[...] = (o / l).astype(jnp.bfloat16)

    return pl.pallas_call(
        body,
        grid=(H, S // BLK),
        in_specs=[
            pl.BlockSpec((BLK, Dh), lambda h, qb: (qb, h)),
            pl.BlockSpec((S, Dh), lambda h, qb: (0, h)),
            pl.BlockSpec((S, Dh), lambda h, qb: (0, h)),
        ],
        out_specs=pl.BlockSpec((BLK, Dh), lambda h, qb: (qb, h)),
        out_shape=jax.ShapeDtypeStruct((S, Dl), jnp.bfloat16),
    )(Q, K, V)


def _matmul_call(A, W, out_dtype=jnp.float32):
    S, D = A.shape
    N = W.shape[1]

    def body(a_ref, w_ref, o_ref):
        o_ref[...] = jnp.dot(
            a_ref[...], w_ref[...], preferred_element_type=jnp.float32
        ).astype(out_dtype)

    return pl.pallas_call(
        body,
        grid=(S // BLK,),
        in_specs=[
            pl.BlockSpec((BLK, D), lambda i: (i, 0)),
            pl.BlockSpec((D, N), lambda i: (0, 0)),
        ],
        out_specs=pl.BlockSpec((BLK, N), lambda i: (i, 0)),
        out_shape=jax.ShapeDtypeStruct((S, N), out_dtype),
    )(A, W)


def _ffn_call(x1, scale_row, shift_row, W1, W2, out_dtype=jnp.float32):
    S, D = x1.shape
    F = W1.shape[1]

    def body(x_ref, sc_ref, sh_ref, w1_ref, w2_ref, o_ref):
        x = _ln(x_ref[...]) * sc_ref[...] + sh_ref[...]
        x = x.astype(jnp.bfloat16)
        h = jnp.dot(x, w1_ref[...], preferred_element_type=jnp.float32)
        h = (h * jax.nn.sigmoid(h)).astype(jnp.bfloat16)
        o_ref[...] = jnp.dot(
            h, w2_ref[...], preferred_element_type=jnp.float32
        ).astype(out_dtype)

    return pl.pallas_call(
        body,
        grid=(S // BLK,),
        in_specs=[
            pl.BlockSpec((BLK, D), lambda i: (i, 0)),
            pl.BlockSpec((1, D), lambda i: (0, 0)),
            pl.BlockSpec((1, D), lambda i: (0, 0)),
            pl.BlockSpec((D, F), lambda i: (0, 0)),
            pl.BlockSpec((F, D), lambda i: (0, 0)),
        ],
        out_specs=pl.BlockSpec((BLK, D), lambda i: (i, 0)),
        out_shape=jax.ShapeDtypeStruct((S, D), out_dtype),
    )(x1, scale_row, shift_row, W1, W2)


def _allreduce(partial, collective_id):
    M, N = partial.shape
    dtype = partial.dtype
    chunk = M // N_DEV
    half = N // 2

    def body(in_ref, out_ref, comm_p, comm_m,
             rs_send_p, rs_recv_p, ag_send_p, ag_recv_p,
             rs_send_m, rs_recv_m, ag_send_m, ag_recv_m):
        my = lax.axis_index("i")
        left = lax.rem(my + N_DEV - 1, N_DEV)
        right = lax.rem(my + 1, N_DEV)

        barrier = pltpu.get_barrier_semaphore()
        for nbr in (left, right):
            pl.semaphore_signal(
                barrier, inc=1, device_id=(nbr,),
                device_id_type=pl.DeviceIdType.MESH,
            )
        pl.semaphore_wait(barrier, 2)

        out_ref[...] = in_ref[...]

        cols_p = pl.ds(0, half)
        cols_m = pl.ds(half, half)

        for t in range(N_DEV - 1):
            cs_p = lax.rem(my - t + N_DEV, N_DEV)
            rdma_p = pltpu.make_async_remote_copy(
                src_ref=out_ref.at[pl.ds(cs_p * chunk, chunk), cols_p],
                dst_ref=comm_p.at[t],
                send_sem=rs_send_p.at[t],
                recv_sem=rs_recv_p.at[t],
                device_id=(right,),
                device_id_type=pl.DeviceIdType.MESH,
            )
            cs_m = lax.rem(my + t, N_DEV)
            rdma_m = pltpu.make_async_remote_copy(
                src_ref=out_ref.at[pl.ds(cs_m * chunk, chunk), cols_m],
                dst_ref=comm_m.at[t],
                send_sem=rs_send_m.at[t],
                recv_sem=rs_recv_m.at[t],
                device_id=(left,),
                device_id_type=pl.DeviceIdType.MESH,
            )
            rdma_p.start()
            rdma_m.start()
            rdma_p.wait()
            cr_p = lax.rem(my - t - 1 + 2 * N_DEV, N_DEV)
            sl = pl.ds(cr_p * chunk, chunk)
            out_ref[sl, cols_p] = out_ref[sl, cols_p] + comm_p[t]
            rdma_m.wait()
            cr_m = lax.rem(my + t + 1, N_DEV)
            sl = pl.ds(cr_m * chunk, chunk)
            out_ref[sl, cols_m] = out_ref[sl, cols_m] + comm_m[t]

        for s in range(N_DEV - 1):
            c_p = lax.rem(my + 1 - s + N_DEV, N_DEV)
            sl_p = pl.ds(c_p * chunk, chunk)
            rdma_p = pltpu.make_async_remote_copy(
                src_ref=out_ref.at[sl_p, cols_p],
                dst_ref=out_ref.at[sl_p, cols_p],
                send_sem=ag_send_p.at[s],
                recv_sem=ag_recv_p.at[s],
                device_id=(right,),
                device_id_type=pl.DeviceIdType.MESH,
            )
            c_m = lax.rem(my - 1 + s + N_DEV, N_DEV)
            sl_m = pl.ds(c_m * chunk, chunk)
            rdma_m = pltpu.make_async_remote_copy(
                src_ref=out_ref.at[sl_m, cols_m],
                dst_ref=out_ref.at[sl_m, cols_m],
                send_sem=ag_send_m.at[s],
                recv_sem=ag_recv_m.at[s],
                device_id=(left,),
                device_id_type=pl.DeviceIdType.MESH,
            )
            rdma_p.start()
            rdma_m.start()
            rdma_p.wait()
            rdma_m.wait()

    n_sem = N_DEV - 1
    return pl.pallas_call(
        body,
        out_shape=jax.ShapeDtypeStruct((M, N), dtype),
        in_specs=[pl.BlockSpec(memory_space=pltpu.VMEM)],
        out_specs=pl.BlockSpec(memory_space=pltpu.VMEM),
        scratch_shapes=[
            pltpu.VMEM((n_sem, chunk, half), dtype),
            pltpu.VMEM((n_sem, chunk, half), dtype),
        ] + [pltpu.SemaphoreType.DMA((n_sem,)) for _ in range(8)],
        compiler_params=pltpu.CompilerParams(collective_id=collective_id),
    )(partial)


def kernel(x, Wq, Wk, Wv, Wo, t_emb, W_mod, W_ff1, W_ff2):
    x0 = x[0]
    mod = jnp.dot(t_emb, W_mod)
    sa, sha, ga, sm, shm, gm = jnp.split(mod, 6, axis=-1)

    bf16 = jnp.bfloat16
    Q, K, V = _qkv_call(
        x0, 1.0 + sa, sha, Wq.astype(bf16), Wk.astype(bf16), Wv.astype(bf16)
    )
    attn = _attn_call(Q, K, V)
    partial = _matmul_call(attn, Wo.astype(bf16), out_dtype=bf16)
    attn_sum = _allreduce(partial, collective_id=0).astype(jnp.float32)

    x1 = x0 + ga * attn_sum
    partial2 = _ffn_call(
        x1, 1.0 + sm, shm, W_ff1.astype(bf16), W_ff2.astype(bf16),
        out_dtype=bf16,
    )
    ffn_sum = _allreduce(partial2, collective_id=1).astype(jnp.float32)

    out = x1 + gm * ffn_sum
    return out[None]


# device time: 432804 ns/iter; 1.1435x vs baseline; 1.1435x over previous
import jax
import jax.numpy as jnp
from jax import lax
from jax.experimental import pallas as pl
from jax.experimental.pallas import tpu as pltpu

N_DEV = 8
SCALE = 0.08838834764831843
EPS = 1e-5
BLK = 512


def _ln(h):
    m = jnp.mean(h, axis=-1, keepdims=True)
    v = jnp.mean((h - m) ** 2, axis=-1, keepdims=True)
    return (h - m) * lax.rsqrt(v + EPS)


def _qkv_call(x0, scale_row, shift_row, Wq, Wk, Wv):
    S, D = x0.shape
    Dl = Wq.shape[1]

    def body(x_ref, sc_ref, sh_ref, wq_ref, wk_ref, wv_ref,
             q_ref, k_ref, v_ref):
        x = _ln(x_ref[...]) * sc_ref[...] + sh_ref[...]
        x = x.astype(jnp.bfloat16)
        q_ref[...] = (
            jnp.dot(x, wq_ref[...], preferred_element_type=jnp.float32)
            * SCALE
        ).astype(jnp.bfloat16)
        k_ref[...] = jnp.dot(
            x, wk_ref[...], preferred_element_type=jnp.float32
        ).astype(jnp.bfloat16)
        v_ref[...] = jnp.dot(
            x, wv_ref[...], preferred_element_type=jnp.float32
        ).astype(jnp.bfloat16)

    w_spec = pl.BlockSpec((D, Dl), lambda i: (0, 0))
    row_spec = pl.BlockSpec((BLK, D), lambda i: (i, 0))
    vec_spec = pl.BlockSpec((1, D), lambda i: (0, 0))
    out_spec = pl.BlockSpec((BLK, Dl), lambda i: (i, 0))
    out_shape = jax.ShapeDtypeStruct((S, Dl), jnp.bfloat16)
    return pl.pallas_call(
        body,
        grid=(S // BLK,),
        in_specs=[row_spec, vec_spec, vec_spec, w_spec, w_spec, w_spec],
        out_specs=(out_spec, out_spec, out_spec),
        out_shape=(out_shape, out_shape, out_shape),
    )(x0, scale_row, shift_row, Wq, Wk, Wv)


def _attn_call(Q, K, V):
    S, Dl = Q.shape
    Dh = 128
    H = Dl // Dh

    def body(q_ref, k_ref, v_ref, o_ref):
        q = q_ref[...]
        k = k_ref[...]
        v = v_ref[...]
        s = lax.dot_general(
            q, k, (((1,), (1,)), ((), ())),
            preferred_element_type=jnp.float32,
        )
        p = jnp.exp(s).astype(jnp.bfloat16)
        l = jnp.sum(p, axis=-1, keepdims=True, dtype=jnp.float32)
        o = jnp.dot(p, v, preferred_element_type=jnp.float32)
        o_ref[...] = (o / l).astype(jnp.bfloat16)

    return pl.pallas_call(
        body,
        grid=(H, S // BLK),
        in_specs=[
            pl.BlockSpec((BLK, Dh), lambda h, qb: (qb, h)),
            pl.BlockSpec((S, Dh), lambda h, qb: (0, h)),
            pl.BlockSpec((S, Dh), lambda h, qb: (0, h)),
        ],
        out_specs=pl.BlockSpec((BLK, Dh), lambda h, qb: (qb, h)),
        out_shape=jax.ShapeDtypeStruct((S, Dl), jnp.bfloat16),
    )(Q, K, V)


def _matmul_call(A, W, out_dtype=jnp.float32):
    S, D = A.shape
    N = W.shape[1]

    def body(a_ref, w_ref, o_ref):
        o_ref[...] = jnp.dot(
            a_ref[...], w_ref[...], preferred_element_type=jnp.float32
        ).astype(out_dtype)

    return pl.pallas_call(
        body,
        grid=(S // BLK,),
        in_specs=[
            pl.BlockSpec((BLK, D), lambda i: (i, 0)),
            pl.BlockSpec((D, N), lambda i: (0, 0)),
        ],
        out_specs=pl.BlockSpec((BLK, N), lambda i: (i, 0)),
        out_shape=jax.ShapeDtypeStruct((S, N), out_dtype),
    )(A, W)


def _ffn_call(x1, scale_row, shift_row, W1, W2, out_dtype=jnp.float32):
    S, D = x1.shape
    F = W1.shape[1]

    def body(x_ref, sc_ref, sh_ref, w1_ref, w2_ref, o_ref):
        x = _ln(x_ref[...]) * sc_ref[...] + sh_ref[...]
        x = x.astype(jnp.bfloat16)
        h = jnp.dot(x, w1_ref[...], preferred_element_type=jnp.float32)
        h = (h * jax.nn.sigmoid(h)).astype(jnp.bfloat16)
        o_ref[...] = jnp.dot(
            h, w2_ref[...], preferred_element_type=jnp.float32
        ).astype(out_dtype)

    return pl.pallas_call(
        body,
        grid=(S // BLK,),
        in_specs=[
            pl.BlockSpec((BLK, D), lambda i: (i, 0)),
            pl.BlockSpec((1, D), lambda i: (0, 0)),
            pl.BlockSpec((1, D), lambda i: (0, 0)),
            pl.BlockSpec((D, F), lambda i: (0, 0)),
            pl.BlockSpec((F, D), lambda i: (0, 0)),
        ],
        out_specs=pl.BlockSpec((BLK, D), lambda i: (i, 0)),
        out_shape=jax.ShapeDtypeStruct((S, D), out_dtype),
    )(x1, scale_row, shift_row, W1, W2)


def _attn_wo_rs_ag_call(Q, K, V, Wo):
    S, Dl = Q.shape
    Dh = 128
    H = Dl // Dh
    chunk = S // N_DEV
    half = Dl // 2
    bf16 = jnp.bfloat16

    def body(q_ref, k_ref, v_ref, wo_ref, out_ref, attn_buf, comm_rs,
             rs_send_p, rs_recv_p, ag_send_p, ag_recv_p,
             ag_send_m, ag_recv_m):
        my = lax.axis_index("i")
        left = lax.rem(my + N_DEV - 1, N_DEV)
        right = lax.rem(my + 1, N_DEV)

        barrier = pltpu.get_barrier_semaphore()
        for nbr in (left, right):
            pl.semaphore_signal(
                barrier, inc=1, device_id=(nbr,),
                device_id_type=pl.DeviceIdType.MESH,
            )
        pl.semaphore_wait(barrier, 2)

        def produce(c):
            rows = pl.ds(c * chunk, chunk)

            def head_body(h, carry):
                hcols = pl.ds(h * Dh, Dh)
                q = q_ref[rows, hcols]
                k = k_ref[:, hcols]
                v = v_ref[:, hcols]
                s = lax.dot_general(
                    q, k, (((1,), (1,)), ((), ())),
                    preferred_element_type=jnp.float32,
                )
                p = jnp.exp(s).astype(bf16)
                l = jnp.sum(p, axis=-1, keepdims=True, dtype=jnp.float32)
                o = jnp.dot(p, v, preferred_element_type=jnp.float32)
                attn_buf[:, hcols] = (o / l).astype(bf16)
                return carry

            lax.fori_loop(0, H, head_body, 0)
            out_ref[rows, :] = jnp.dot(
                attn_buf[...], wo_ref[...], preferred_element_type=jnp.float32
            ).astype(bf16)

        cols_p = pl.ds(0, half)
        cols_m = pl.ds(half, half)

        produce(lax.rem(my, N_DEV))
        for t in range(N_DEV - 1):
            cs = lax.rem(my - t + N_DEV, N_DEV)
            rdma = pltpu.make_async_remote_copy(
                src_ref=out_ref.at[pl.ds(cs * chunk, chunk), :],
                dst_ref=comm_rs.at[t],
                send_sem=rs_send_p.at[t],
                recv_sem=rs_recv_p.at[t],
                device_id=(right,),
                device_id_type=pl.DeviceIdType.MESH,
            )
            rdma.start()
            produce(lax.rem(my - t - 1 + 2 * N_DEV, N_DEV))
            rdma.wait()
            cr = lax.rem(my - t - 1 + 2 * N_DEV, N_DEV)
            sl = pl.ds(cr * chunk, chunk)
            out_ref[sl, :] = out_ref[sl, :] + comm_rs[t]

        for s in range(N_DEV - 1):
            c_p = lax.rem(my + 1 - s + N_DEV, N_DEV)
            sl_p = pl.ds(c_p * chunk, chunk)
            rdma_p = pltpu.make_async_remote_copy(
                src_ref=out_ref.at[sl_p, cols_p],
                dst_ref=out_ref.at[sl_p, cols_p],
                send_sem=ag_send_p.at[s],
                recv_sem=ag_recv_p.at[s],
                device_id=(right,),
                device_id_type=pl.DeviceIdType.MESH,
            )
            c_m = lax.rem(my + 1 + s, N_DEV)
            sl_m = pl.ds(c_m * chunk, chunk)
            rdma_m = pltpu.make_async_remote_copy(
                src_ref=out_ref.at[sl_m, cols_m],
                dst_ref=out_ref.at[sl_m, cols_m],
                send_sem=ag_send_m.at[s],
                recv_sem=ag_recv_m.at[s],
                device_id=(left,),
                device_id_type=pl.DeviceIdType.MESH,
            )
            rdma_p.start()
            rdma_m.start()
            rdma_p.wait()
            rdma_m.wait()

    n_sem = N_DEV - 1
    return pl.pallas_call(
        body,
        out_shape=jax.ShapeDtypeStruct((S, Dl), bf16),
        in_specs=[pl.BlockSpec(memory_space=pltpu.VMEM)] * 4,
        out_specs=pl.BlockSpec(memory_space=pltpu.VMEM),
        scratch_shapes=[
            pltpu.VMEM((chunk, Dl), bf16),
            pltpu.VMEM((n_sem, chunk, Dl), bf16),
        ] + [pltpu.SemaphoreType.DMA((n_sem,)) for _ in range(6)],
        compiler_params=pltpu.CompilerParams(collective_id=0),
    )(Q, K, V, Wo)

def _allreduce(partial, collective_id):
    M, N = partial.shape
    dtype = partial.dtype
    chunk = M // N_DEV
    half = N // 2

    def body(in_ref, out_ref, comm_p, comm_m,
             rs_send_p, rs_recv_p, ag_send_p, ag_recv_p,
             rs_send_m, rs_recv_m, ag_send_m, ag_recv_m):
        my = lax.axis_index("i")
        left = lax.rem(my + N_DEV - 1, N_DEV)
        right = lax.rem(my + 1, N_DEV)

        barrier = pltpu.get_barrier_semaphore()
        for nbr in (left, right):
            pl.semaphore_signal(
                barrier, inc=1, device_id=(nbr,),
                device_id_type=pl.DeviceIdType.MESH,
            )
        pl.semaphore_wait(barrier, 2)

        out_ref[...] = in_ref[...]

        cols_p = pl.ds(0, half)
        cols_m = pl.ds(half, half)

        for t in range(N_DEV - 1):
            cs_p = lax.rem(my - t + N_DEV, N_DEV)
            rdma_p = pltpu.make_async_remote_copy(
                src_ref=out_ref.at[pl.ds(cs_p * chunk, chunk), cols_p],
                dst_ref=comm_p.at[t],
                send_sem=rs_send_p.at[t],
                recv_sem=rs_recv_p.at[t],
                device_id=(right,),
                device_id_type=pl.DeviceIdType.MESH,
            )
            cs_m = lax.rem(my + t, N_DEV)
            rdma_m = pltpu.make_async_remote_copy(
                src_ref=out_ref.at[pl.ds(cs_m * chunk, chunk), cols_m],
                dst_ref=comm_m.at[t],
                send_sem=rs_send_m.at[t],
                recv_sem=rs_recv_m.at[t],
                device_id=(left,),
                device_id_type=pl.DeviceIdType.MESH,
            )
            rdma_p.start()
            rdma_m.start()
            rdma_p.wait()
            cr_p = lax.rem(my - t - 1 + 2 * N_DEV, N_DEV)
            sl = pl.ds(cr_p * chunk, chunk)
            out_ref[sl, cols_p] = out_ref[sl, cols_p] + comm_p[t]
            rdma_m.wait()
            cr_m = lax.rem(my + t + 1, N_DEV)
            sl = pl.ds(cr_m * chunk, chunk)
            out_ref[sl, cols_m] = out_ref[sl, cols_m] + comm_m[t]

        for s in range(N_DEV - 1):
            c_p = lax.rem(my + 1 - s + N_DEV, N_DEV)
            sl_p = pl.ds(c_p * chunk, chunk)
            rdma_p = pltpu.make_async_remote_copy(
                src_ref=out_ref.at[sl_p, cols_p],
                dst_ref=out_ref.at[sl_p, cols_p],
                send_sem=ag_send_p.at[s],
                recv_sem=ag_recv_p.at[s],
                device_id=(right,),
                device_id_type=pl.DeviceIdType.MESH,
            )
            c_m = lax.rem(my - 1 + s + N_DEV, N_DEV)
            sl_m = pl.ds(c_m * chunk, chunk)
            rdma_m = pltpu.make_async_remote_copy(
                src_ref=out_ref.at[sl_m, cols_m],
                dst_ref=out_ref.at[sl_m, cols_m],
                send_sem=ag_send_m.at[s],
                recv_sem=ag_recv_m.at[s],
                device_id=(left,),
                device_id_type=pl.DeviceIdType.MESH,
            )
            rdma_p.start()
            rdma_m.start()
            rdma_p.wait()
            rdma_m.wait()

    n_sem = N_DEV - 1
    return pl.pallas_call(
        body,
        out_shape=jax.ShapeDtypeStruct((M, N), dtype),
        in_specs=[pl.BlockSpec(memory_space=pltpu.VMEM)],
        out_specs=pl.BlockSpec(memory_space=pltpu.VMEM),
        scratch_shapes=[
            pltpu.VMEM((n_sem, chunk, half), dtype),
            pltpu.VMEM((n_sem, chunk, half), dtype),
        ] + [pltpu.SemaphoreType.DMA((n_sem,)) for _ in range(8)],
        compiler_params=pltpu.CompilerParams(collective_id=collective_id),
    )(partial)


def kernel(x, Wq, Wk, Wv, Wo, t_emb, W_mod, W_ff1, W_ff2):
    x0 = x[0]
    mod = jnp.dot(t_emb, W_mod)
    sa, sha, ga, sm, shm, gm = jnp.split(mod, 6, axis=-1)

    bf16 = jnp.bfloat16
    Q, K, V = _qkv_call(
        x0, 1.0 + sa, sha, Wq.astype(bf16), Wk.astype(bf16), Wv.astype(bf16)
    )
    attn_sum = _attn_wo_rs_ag_call(Q, K, V, Wo.astype(bf16)).astype(
        jnp.float32
    )

    x1 = x0 + ga * attn_sum
    partial2 = _ffn_call(
        x1, 1.0 + sm, shm, W_ff1.astype(bf16), W_ff2.astype(bf16),
        out_dtype=bf16,
    )
    ffn_sum = _allreduce(partial2, collective_id=1).astype(jnp.float32)

    out = x1 + gm * ffn_sum
    return out[None]
